# baseline (speedup 1.0000x reference)
import jax
import jax.numpy as jnp
from jax import lax
from jax.experimental import pallas as pl

_NUM_BINS = 15
_GAMMA_INITIAL = 1.0
_ROWS_PER_BLOCK = 4096


def _body(x_ref, t_ref, out_ref):
    x = x_ref[...]
    t = t_ref[...]
    r, c = x.shape
    m = jnp.max(x, axis=1, keepdims=True)
    e = jnp.exp(x - m)
    s = jnp.sum(e, axis=1, keepdims=True)
    lse = m + jnp.log(s)
    cols = lax.broadcasted_iota(jnp.int32, (r, c), 1)
    xt = jnp.sum(jnp.where(cols == t, x, 0.0), axis=1, keepdims=True)
    logpt = xt - lse
    pt = jnp.exp(logpt)
    loss = -(1.0 - pt + 1e-20) * logpt
    part = jnp.sum(loss).reshape(1, 1)

    @pl.when(pl.program_id(0) == 0)
    def _():
        out_ref[...] = jnp.zeros((1, 1), jnp.float32)

    out_ref[...] += part


def kernel(input, target):
    batch, ncls = input.shape
    grid = batch // _ROWS_PER_BLOCK
    t2 = target.reshape(batch, 1).astype(jnp.int32)
    out = pl.pallas_call(
        _body,
        grid=(grid,),
        in_specs=[
            pl.BlockSpec((_ROWS_PER_BLOCK, ncls), lambda i: (i, 0)),
            pl.BlockSpec((_ROWS_PER_BLOCK, 1), lambda i: (i, 0)),
        ],
        out_specs=pl.BlockSpec((1, 1), lambda i: (0, 0)),
        out_shape=jax.ShapeDtypeStruct((1, 1), jnp.float32),
    )(input, t2)
    return out[0, 0]
